# baseline (speedup 1.0000x reference)
import functools

import jax
import jax.numpy as jnp
from jax import lax
from jax.experimental import pallas as pl
from jax.experimental.pallas import tpu as pltpu
from jax.experimental.pallas import tpu_sc as plsc

B = 4096
L = 20
V = 1000
VCH = 40
NVC = V // VCH
NSLAB = L * NVC

_info = plsc.get_sparse_core_info()
NC, NS, LANES = _info.num_cores, _info.num_subcores, _info.num_lanes
NW = NC * NS
BPW = B // NW
NGRP = BPW // LANES


def _scatter_slab(buf, xv, l, voff, val):
    lane = lax.iota(jnp.int32, LANES)
    vvec = jnp.full((LANES,), val, jnp.float32)
    for k in range(NGRP):
        xval = xv[l, pl.ds(k * LANES, LANES)]
        local = xval - voff
        mask = (local >= 0) & (local < VCH)
        plsc.store_scatter(buf, [local, lane + (k * LANES)], vvec, mask=mask)


def _onehot_body(xt_hbm, zeros_hbm, out_hbm, xv, buf0, buf1, sem0, sem1):
    bufs = (buf0, buf1)
    sems = (sem0, sem1)
    wid = lax.axis_index("s") * NC + lax.axis_index("c")
    base = wid * BPW

    z0 = pltpu.make_async_copy(zeros_hbm, buf0, sem0)
    z1 = pltpu.make_async_copy(zeros_hbm, buf1, sem1)
    z0.start()
    z1.start()
    pltpu.sync_copy(xt_hbm.at[:, pl.ds(base, BPW)], xv)
    z0.wait()
    z1.wait()

    def slab_lvc(s):
        l = s // NVC
        vc = s - l * NVC
        return l, vc * VCH

    def start_slab(b, s):
        l, voff = slab_lvc(s)
        _scatter_slab(bufs[b], xv, l, voff, 1.0)
        pltpu.make_async_copy(
            bufs[b],
            out_hbm.at[l, pl.ds(voff, VCH), pl.ds(base, BPW)],
            sems[b],
        ).start()

    def finish_slab(b, s):
        l, voff = slab_lvc(s)
        pltpu.make_async_copy(
            bufs[b],
            out_hbm.at[l, pl.ds(voff, VCH), pl.ds(base, BPW)],
            sems[b],
        ).wait()
        _scatter_slab(bufs[b], xv, l, voff, 0.0)

    for b in range(2):
        start_slab(b, jnp.int32(b))

    def pair_body(g, _):
        for b in range(2):
            s = 2 * g + b
            finish_slab(b, s - 2)
            start_slab(b, s)
        return 0

    lax.fori_loop(1, NSLAB // 2, pair_body, 0)

    for b in range(2):
        l, voff = slab_lvc(jnp.int32(NSLAB - 2 + b))
        pltpu.make_async_copy(
            bufs[b],
            out_hbm.at[l, pl.ds(voff, VCH), pl.ds(base, BPW)],
            sems[b],
        ).wait()


@jax.jit
def _onehot(xt, zeros):
    mesh = plsc.VectorSubcoreMesh(core_axis_name="c", subcore_axis_name="s")
    f = functools.partial(
        pl.kernel,
        out_type=jax.ShapeDtypeStruct((L, V, B), jnp.float32),
        mesh=mesh,
        scratch_types=[
            pltpu.VMEM((L, BPW), jnp.int32),
            pltpu.VMEM((VCH, BPW), jnp.float32),
            pltpu.VMEM((VCH, BPW), jnp.float32),
            pltpu.SemaphoreType.DMA,
            pltpu.SemaphoreType.DMA,
        ],
        compiler_params=pltpu.CompilerParams(needs_layout_passes=False),
    )(_onehot_body)
    return f(xt, zeros)


def kernel(x):
    zeros = jnp.zeros((VCH, BPW), jnp.float32)
    out = _onehot(x.T, zeros)
    return out.transpose(2, 0, 1)


# speedup vs baseline: 1.0177x; 1.0177x over previous
import functools

import jax
import jax.numpy as jnp
from jax import lax
from jax.experimental import pallas as pl
from jax.experimental.pallas import tpu as pltpu
from jax.experimental.pallas import tpu_sc as plsc

B = 4096
L = 20
V = 1000
VCH = 40
NVC = V // VCH
NSLAB = L * NVC

_info = plsc.get_sparse_core_info()
NC, NS, LANES = _info.num_cores, _info.num_subcores, _info.num_lanes
NW = NC * NS
BPW = B // NW
NGRP = BPW // LANES


def _scatter_slab(buf, xv, l, voff, val):
    lane = lax.iota(jnp.int32, LANES)
    vvec = jnp.full((LANES,), val, jnp.float32)
    for k in range(NGRP):
        xval = xv[l, pl.ds(k * LANES, LANES)]
        local = xval - voff
        mask = (local >= 0) & (local < VCH)
        plsc.store_scatter(buf, [local, lane + (k * LANES)], vvec, mask=mask)


def _onehot_body(xt_hbm, zeros_hbm, out_hbm, xv, buf0, buf1, sem0, sem1):
    bufs = (buf0, buf1)
    sems = (sem0, sem1)
    wid = lax.axis_index("s") * NC + lax.axis_index("c")
    base = wid * BPW

    z0 = pltpu.make_async_copy(zeros_hbm, buf0, sem0)
    z1 = pltpu.make_async_copy(zeros_hbm, buf1, sem1)
    z0.start()
    z1.start()
    pltpu.sync_copy(xt_hbm.at[wid], xv)
    z0.wait()
    z1.wait()

    def slab_lvc(s):
        l = s // NVC
        vc = s - l * NVC
        return l, vc * VCH

    def start_slab(b, s):
        l, voff = slab_lvc(s)
        _scatter_slab(bufs[b], xv, l, voff, 1.0)
        pltpu.make_async_copy(
            bufs[b],
            out_hbm.at[l, pl.ds(voff, VCH), pl.ds(base, BPW)],
            sems[b],
        ).start()

    def finish_slab(b, s):
        l, voff = slab_lvc(s)
        pltpu.make_async_copy(
            bufs[b],
            out_hbm.at[l, pl.ds(voff, VCH), pl.ds(base, BPW)],
            sems[b],
        ).wait()
        _scatter_slab(bufs[b], xv, l, voff, 0.0)

    for b in range(2):
        start_slab(b, jnp.int32(b))

    def pair_body(g, _):
        for b in range(2):
            s = 2 * g + b
            finish_slab(b, s - 2)
            start_slab(b, s)
        return 0

    lax.fori_loop(1, NSLAB // 2, pair_body, 0)

    for b in range(2):
        l, voff = slab_lvc(jnp.int32(NSLAB - 2 + b))
        pltpu.make_async_copy(
            bufs[b],
            out_hbm.at[l, pl.ds(voff, VCH), pl.ds(base, BPW)],
            sems[b],
        ).wait()


@jax.jit
def _onehot(xt, zeros):
    mesh = plsc.VectorSubcoreMesh(core_axis_name="c", subcore_axis_name="s")
    f = functools.partial(
        pl.kernel,
        out_type=jax.ShapeDtypeStruct((L, V, B), jnp.float32),
        mesh=mesh,
        scratch_types=[
            pltpu.VMEM((L, BPW), jnp.int32),
            pltpu.VMEM((VCH, BPW), jnp.float32),
            pltpu.VMEM((VCH, BPW), jnp.float32),
            pltpu.SemaphoreType.DMA,
            pltpu.SemaphoreType.DMA,
        ],
        compiler_params=pltpu.CompilerParams(needs_layout_passes=False),
    )(_onehot_body)
    return f(xt, zeros)


def kernel(x):
    xprep = x.reshape(NW, BPW, L).transpose(0, 2, 1)
    zeros = jnp.zeros((VCH, BPW), jnp.float32)
    out = _onehot(xprep, zeros)
    return out.transpose(2, 0, 1)
